# baseline (device time: 154715 ns/iter reference)
import jax
import jax.numpy as jnp
from jax import lax
from jax.experimental import pallas as pl
from jax.experimental.pallas import tpu as pltpu

N_DEV = 4
Q_BLOCK = 512


def kernel(q, k, v):
    s_per, d = q.shape
    scale = 1.0 / (d ** 0.5)
    n_qblk = s_per // Q_BLOCK

    def body(q_ref, k_ref, v_ref, out_ref, k_comm, v_comm, l_ref,
             k_send, k_recv, v_send, v_recv):
        my_pos = lax.axis_index("i")
        left = (my_pos - 1) % N_DEV
        right = (my_pos + 1) % N_DEV

        barrier_sem = pltpu.get_barrier_semaphore()
        for nbr in [left, right]:
            pl.semaphore_signal(
                barrier_sem, inc=1,
                device_id=(nbr,), device_id_type=pl.DeviceIdType.MESH,
            )
        pl.semaphore_wait(barrier_sem, 2)

        for h in range(N_DEV):
            rdmas = []
            if h < N_DEV - 1:
                for src, comm, ssem, rsem in (
                    (k_ref if h == 0 else k_comm.at[h - 1], k_comm, k_send, k_recv),
                    (v_ref if h == 0 else v_comm.at[h - 1], v_comm, v_send, v_recv),
                ):
                    rdma = pltpu.make_async_remote_copy(
                        src_ref=src,
                        dst_ref=comm.at[h],
                        send_sem=ssem.at[h],
                        recv_sem=rsem.at[h],
                        device_id=(right,),
                        device_id_type=pl.DeviceIdType.MESH,
                    )
                    rdma.start()
                    rdmas.append(rdma)

            kblk = k_ref[:, :] if h == 0 else k_comm[h - 1]
            vblk = v_ref[:, :] if h == 0 else v_comm[h - 1]
            for qi in range(n_qblk):
                rows = pl.ds(qi * Q_BLOCK, Q_BLOCK)
                qb = q_ref[rows, :]
                s = lax.dot_general(
                    qb, kblk, (((1,), (1,)), ((), ())),
                    preferred_element_type=jnp.float32,
                ) * scale
                p = jnp.exp(s)
                pv = lax.dot_general(
                    p, vblk, (((1,), (0,)), ((), ())),
                    preferred_element_type=jnp.float32,
                )
                rsum = jnp.sum(p, axis=1, keepdims=True)
                if h == 0:
                    out_ref[rows, :] = pv
                    l_ref[rows, :] = rsum
                else:
                    out_ref[rows, :] = out_ref[rows, :] + pv
                    l_ref[rows, :] = l_ref[rows, :] + rsum

            for rdma in rdmas:
                rdma.wait()

        out_ref[:, :] = out_ref[:, :] / l_ref[:, :]

    return pl.pallas_call(
        body,
        out_shape=jax.ShapeDtypeStruct((s_per, d), jnp.float32),
        in_specs=[pl.BlockSpec(memory_space=pltpu.VMEM)] * 3,
        out_specs=pl.BlockSpec(memory_space=pltpu.VMEM),
        scratch_shapes=[
            pltpu.VMEM((N_DEV - 1, s_per, d), jnp.float32),
            pltpu.VMEM((N_DEV - 1, s_per, d), jnp.float32),
            pltpu.VMEM((s_per, 1), jnp.float32),
            pltpu.SemaphoreType.DMA((N_DEV - 1,)),
            pltpu.SemaphoreType.DMA((N_DEV - 1,)),
            pltpu.SemaphoreType.DMA((N_DEV - 1,)),
            pltpu.SemaphoreType.DMA((N_DEV - 1,)),
        ],
        compiler_params=pltpu.CompilerParams(collective_id=0),
    )(q, k, v)


# device time: 86811 ns/iter; 1.7822x vs baseline; 1.7822x over previous
import jax
import jax.numpy as jnp
from jax import lax
from jax.experimental import pallas as pl
from jax.experimental.pallas import tpu as pltpu

N_DEV = 4
Q_BLOCK = 512


def kernel(q, k, v):
    s_per, d = q.shape
    half = s_per // 2
    scale = 1.0 / (d ** 0.5)
    n_qblk = s_per // Q_BLOCK
    n_hops = N_DEV - 1

    def body(q_ref, k_ref, v_ref, out_ref,
             k_cr, v_cr, k_cl, v_cl, l_ref,
             k_cr_s, k_cr_r, v_cr_s, v_cr_r,
             k_cl_s, k_cl_r, v_cl_s, v_cl_r):
        my_pos = lax.axis_index("i")
        left = (my_pos - 1) % N_DEV
        right = (my_pos + 1) % N_DEV

        barrier_sem = pltpu.get_barrier_semaphore()
        for nbr in [left, right]:
            pl.semaphore_signal(
                barrier_sem, inc=1,
                device_id=(nbr,), device_id_type=pl.DeviceIdType.MESH,
            )
        pl.semaphore_wait(barrier_sem, 2)

        top = pl.ds(0, half)
        bot = pl.ds(half, half)

        for h in range(N_DEV):
            rdmas = []
            if h < n_hops:
                for src, comm, ssem, rsem, dst_dev in (
                    (k_ref.at[top] if h == 0 else k_cr.at[h - 1], k_cr, k_cr_s, k_cr_r, right),
                    (v_ref.at[top] if h == 0 else v_cr.at[h - 1], v_cr, v_cr_s, v_cr_r, right),
                    (k_ref.at[bot] if h == 0 else k_cl.at[h - 1], k_cl, k_cl_s, k_cl_r, left),
                    (v_ref.at[bot] if h == 0 else v_cl.at[h - 1], v_cl, v_cl_s, v_cl_r, left),
                ):
                    rdma = pltpu.make_async_remote_copy(
                        src_ref=src,
                        dst_ref=comm.at[h],
                        send_sem=ssem.at[h],
                        recv_sem=rsem.at[h],
                        device_id=(dst_dev,),
                        device_id_type=pl.DeviceIdType.MESH,
                    )
                    rdma.start()
                    rdmas.append(rdma)

            if h == 0:
                halves = [(k_ref[top, :], v_ref[top, :]),
                          (k_ref[bot, :], v_ref[bot, :])]
            else:
                halves = [(k_cr[h - 1], v_cr[h - 1]),
                          (k_cl[h - 1], v_cl[h - 1])]
            for qi in range(n_qblk):
                rows = pl.ds(qi * Q_BLOCK, Q_BLOCK)
                qb = q_ref[rows, :]
                pv = None
                rsum = None
                for kblk, vblk in halves:
                    s = lax.dot_general(
                        qb, kblk, (((1,), (1,)), ((), ())),
                        preferred_element_type=jnp.float32,
                    ) * scale
                    p = jnp.exp(s)
                    pv_h = lax.dot_general(
                        p, vblk, (((1,), (0,)), ((), ())),
                        preferred_element_type=jnp.float32,
                    )
                    rsum_h = jnp.sum(p, axis=1, keepdims=True)
                    pv = pv_h if pv is None else pv + pv_h
                    rsum = rsum_h if rsum is None else rsum + rsum_h
                if h == 0:
                    out_ref[rows, :] = pv
                    l_ref[rows, :] = rsum
                else:
                    out_ref[rows, :] = out_ref[rows, :] + pv
                    l_ref[rows, :] = l_ref[rows, :] + rsum

            for rdma in rdmas:
                rdma.wait()

        out_ref[:, :] = out_ref[:, :] / l_ref[:, :]

    return pl.pallas_call(
        body,
        out_shape=jax.ShapeDtypeStruct((s_per, d), jnp.float32),
        in_specs=[pl.BlockSpec(memory_space=pltpu.VMEM)] * 3,
        out_specs=pl.BlockSpec(memory_space=pltpu.VMEM),
        scratch_shapes=[
            pltpu.VMEM((n_hops, half, d), jnp.float32),
            pltpu.VMEM((n_hops, half, d), jnp.float32),
            pltpu.VMEM((n_hops, half, d), jnp.float32),
            pltpu.VMEM((n_hops, half, d), jnp.float32),
            pltpu.VMEM((s_per, 1), jnp.float32),
            pltpu.SemaphoreType.DMA((n_hops,)),
            pltpu.SemaphoreType.DMA((n_hops,)),
            pltpu.SemaphoreType.DMA((n_hops,)),
            pltpu.SemaphoreType.DMA((n_hops,)),
            pltpu.SemaphoreType.DMA((n_hops,)),
            pltpu.SemaphoreType.DMA((n_hops,)),
            pltpu.SemaphoreType.DMA((n_hops,)),
            pltpu.SemaphoreType.DMA((n_hops,)),
        ],
        compiler_params=pltpu.CompilerParams(collective_id=0),
    )(q, k, v)


# device time: 54252 ns/iter; 2.8518x vs baseline; 1.6001x over previous
import jax
import jax.numpy as jnp
from jax import lax
from jax.experimental import pallas as pl
from jax.experimental.pallas import tpu as pltpu

N_DEV = 4
Q_BLOCK = 512


def kernel(q, k, v):
    s_per, d = q.shape
    half = s_per // 2
    scale = 1.0 / (d ** 0.5)
    n_qblk = s_per // Q_BLOCK
    n_hops = N_DEV - 1

    def body(q_ref, k_ref, v_ref, out_ref,
             k_cr, v_cr, k_cl, v_cl,
             k_st_t, v_st_t, k_st_b, v_st_b, l_ref,
             k_cr_s, k_cr_r, v_cr_s, v_cr_r,
             k_cl_s, k_cl_r, v_cl_s, v_cl_r):
        my_pos = lax.axis_index("i")
        left = (my_pos - 1) % N_DEV
        right = (my_pos + 1) % N_DEV

        top = pl.ds(0, half)
        bot = pl.ds(half, half)

        k_st_t[:, :] = k_ref[top, :].astype(jnp.bfloat16)
        v_st_t[:, :] = v_ref[top, :].astype(jnp.bfloat16)
        k_st_b[:, :] = k_ref[bot, :].astype(jnp.bfloat16)
        v_st_b[:, :] = v_ref[bot, :].astype(jnp.bfloat16)

        barrier_sem = pltpu.get_barrier_semaphore()
        for nbr in [left, right]:
            pl.semaphore_signal(
                barrier_sem, inc=1,
                device_id=(nbr,), device_id_type=pl.DeviceIdType.MESH,
            )
        pl.semaphore_wait(barrier_sem, 2)

        for h in range(N_DEV):
            rdmas = []
            if h < n_hops:
                for src, comm, ssem, rsem, dst_dev in (
                    (k_st_t if h == 0 else k_cr.at[h - 1], k_cr, k_cr_s, k_cr_r, right),
                    (v_st_t if h == 0 else v_cr.at[h - 1], v_cr, v_cr_s, v_cr_r, right),
                    (k_st_b if h == 0 else k_cl.at[h - 1], k_cl, k_cl_s, k_cl_r, left),
                    (v_st_b if h == 0 else v_cl.at[h - 1], v_cl, v_cl_s, v_cl_r, left),
                ):
                    rdma = pltpu.make_async_remote_copy(
                        src_ref=src,
                        dst_ref=comm.at[h],
                        send_sem=ssem.at[h],
                        recv_sem=rsem.at[h],
                        device_id=(dst_dev,),
                        device_id_type=pl.DeviceIdType.MESH,
                    )
                    rdma.start()
                    rdmas.append(rdma)

            if h == 0:
                halves = [(k_ref[top, :], v_ref[top, :]),
                          (k_ref[bot, :], v_ref[bot, :])]
            else:
                halves = [(k_cr[h - 1].astype(jnp.float32),
                           v_cr[h - 1].astype(jnp.float32)),
                          (k_cl[h - 1].astype(jnp.float32),
                           v_cl[h - 1].astype(jnp.float32))]
            for qi in range(n_qblk):
                rows = pl.ds(qi * Q_BLOCK, Q_BLOCK)
                qb = q_ref[rows, :]
                pv = None
                rsum = None
                for kblk, vblk in halves:
                    s = lax.dot_general(
                        qb, kblk, (((1,), (1,)), ((), ())),
                        preferred_element_type=jnp.float32,
                    ) * scale
                    p = jnp.exp(s)
                    pv_h = lax.dot_general(
                        p, vblk, (((1,), (0,)), ((), ())),
                        preferred_element_type=jnp.float32,
                    )
                    rsum_h = jnp.sum(p, axis=1, keepdims=True)
                    pv = pv_h if pv is None else pv + pv_h
                    rsum = rsum_h if rsum is None else rsum + rsum_h
                if h == 0:
                    out_ref[rows, :] = pv
                    l_ref[rows, :] = rsum
                else:
                    out_ref[rows, :] = out_ref[rows, :] + pv
                    l_ref[rows, :] = l_ref[rows, :] + rsum

            for rdma in rdmas:
                rdma.wait()

        out_ref[:, :] = out_ref[:, :] / l_ref[:, :]

    return pl.pallas_call(
        body,
        out_shape=jax.ShapeDtypeStruct((s_per, d), jnp.float32),
        in_specs=[pl.BlockSpec(memory_space=pltpu.VMEM)] * 3,
        out_specs=pl.BlockSpec(memory_space=pltpu.VMEM),
        scratch_shapes=[
            pltpu.VMEM((n_hops, half, d), jnp.bfloat16),
            pltpu.VMEM((n_hops, half, d), jnp.bfloat16),
            pltpu.VMEM((n_hops, half, d), jnp.bfloat16),
            pltpu.VMEM((n_hops, half, d), jnp.bfloat16),
            pltpu.VMEM((half, d), jnp.bfloat16),
            pltpu.VMEM((half, d), jnp.bfloat16),
            pltpu.VMEM((half, d), jnp.bfloat16),
            pltpu.VMEM((half, d), jnp.bfloat16),
            pltpu.VMEM((s_per, 1), jnp.float32),
            pltpu.SemaphoreType.DMA((n_hops,)),
            pltpu.SemaphoreType.DMA((n_hops,)),
            pltpu.SemaphoreType.DMA((n_hops,)),
            pltpu.SemaphoreType.DMA((n_hops,)),
            pltpu.SemaphoreType.DMA((n_hops,)),
            pltpu.SemaphoreType.DMA((n_hops,)),
            pltpu.SemaphoreType.DMA((n_hops,)),
            pltpu.SemaphoreType.DMA((n_hops,)),
        ],
        compiler_params=pltpu.CompilerParams(collective_id=0),
    )(q, k, v)
